# Initial kernel scaffold; baseline (speedup 1.0000x reference)
import jax
import jax.numpy as jnp
from jax.experimental import pallas as pl


def kernel(idx, A):
    raise NotImplementedError("write your pallas kernel here")


# speedup vs baseline: 7.2089x; 7.2089x over previous
import functools

import jax
import jax.numpy as jnp
from jax.experimental import pallas as pl
from jax.experimental.pallas import tpu as pltpu

ALPHA_C = 3.0
K_C = 30
_HI_BITS = 0x3F800001
_BIG = 0x7FFFFFFF


def _body(a_ref, o_ref, u_ref, mi_ref, *, n_cols, k):
    x = a_ref[...]
    adj = jnp.tanh(ALPHA_C * x)
    o_ref[...] = adj
    u = jax.lax.bitcast_convert_type(jnp.abs(adj), jnp.int32)
    u_ref[...] = u
    br = x.shape[0]

    def vstep(_, carry):
        lo, hi = carry
        mid = lo + ((hi - lo) >> 1)
        c = jnp.sum((u_ref[...] >= mid).astype(jnp.int32), axis=1,
                    keepdims=True)
        pred = c >= k
        return (jnp.where(pred, mid, lo), jnp.where(pred, hi, mid))

    lo0 = jnp.zeros((br, 1), jnp.int32)
    hi0 = jnp.full((br, 1), _HI_BITS, jnp.int32)
    t_bits, _ = jax.lax.fori_loop(0, 30, vstep, (lo0, hi0))

    uu = u_ref[...]
    cg = jnp.sum((uu > t_bits).astype(jnp.int32), axis=1, keepdims=True)
    m = k - cg

    iota = jax.lax.broadcasted_iota(jnp.int32, (br, n_cols), 1)
    mi_ref[...] = jnp.where(uu == t_bits, iota, _BIG)

    def istep(_, carry):
        lo, hi = carry
        mid = lo + ((hi - lo) >> 1)
        c = jnp.sum((mi_ref[...] <= mid).astype(jnp.int32), axis=1,
                    keepdims=True)
        pred = c >= m
        return (jnp.where(pred, lo, mid), jnp.where(pred, mid, hi))

    lo0j = jnp.full((br, 1), -1, jnp.int32)
    hi0j = jnp.full((br, 1), n_cols - 1, jnp.int32)
    _, j_cut = jax.lax.fori_loop(0, 14, istep, (lo0j, hi0j))

    keep = (uu > t_bits) | (mi_ref[...] <= j_cut)
    o_ref[...] = jnp.where(keep, o_ref[...], 0.0)


def kernel(idx, A):
    n, n_cols = A.shape
    del idx
    br = next(b for b in (80, 40, 16, 8, 1) if n % b == 0)
    body = functools.partial(_body, n_cols=n_cols, k=K_C)
    return pl.pallas_call(
        body,
        grid=(n // br,),
        in_specs=[pl.BlockSpec((br, n_cols), lambda i: (i, 0))],
        out_specs=pl.BlockSpec((br, n_cols), lambda i: (i, 0)),
        out_shape=jax.ShapeDtypeStruct((n, n_cols), jnp.float32),
        scratch_shapes=[
            pltpu.VMEM((br, n_cols), jnp.int32),
            pltpu.VMEM((br, n_cols), jnp.int32),
        ],
    )(A)


# speedup vs baseline: 8.7715x; 1.2168x over previous
import functools

import jax
import jax.numpy as jnp
from jax.experimental import pallas as pl
from jax.experimental.pallas import tpu as pltpu

ALPHA_C = 3.0
K_C = 30
_HI_BITS = 0x3F800001
_BIG = 0x7FFFFFFF


def _body(a_ref, o_ref, u_ref, *, n_cols, k):
    x = a_ref[...]
    adj = jnp.tanh(ALPHA_C * x)
    o_ref[...] = adj
    u = jax.lax.bitcast_convert_type(jnp.abs(adj), jnp.int32)
    u_ref[...] = u
    br = x.shape[0]

    def vstep(_, carry):
        lo, hi = carry
        mid = lo + ((hi - lo) >> 1)
        c = jnp.sum((u_ref[...] >= mid).astype(jnp.int32), axis=1,
                    keepdims=True)
        pred = c >= k
        return (jnp.where(pred, mid, lo), jnp.where(pred, hi, mid))

    lo0 = jnp.zeros((br, 1), jnp.int32)
    hi0 = jnp.full((br, 1), _HI_BITS, jnp.int32)
    t_bits, _ = jax.lax.fori_loop(0, 30, vstep, (lo0, hi0))

    uu = u_ref[...]
    iota = jax.lax.broadcasted_iota(jnp.int32, (br, n_cols), 1)
    u_ref[...] = jnp.where(uu > t_bits, -1,
                           jnp.where(uu == t_bits, iota, _BIG))

    def istep(_, carry):
        lo, hi = carry
        mid = lo + ((hi - lo) >> 1)
        c = jnp.sum((u_ref[...] <= mid).astype(jnp.int32), axis=1,
                    keepdims=True)
        pred = c >= k
        return (jnp.where(pred, lo, mid), jnp.where(pred, mid, hi))

    lo0j = jnp.full((br, 1), -1, jnp.int32)
    hi0j = jnp.full((br, 1), n_cols - 1, jnp.int32)
    _, j_cut = jax.lax.fori_loop(0, 14, istep, (lo0j, hi0j))

    o_ref[...] = jnp.where(u_ref[...] <= j_cut, o_ref[...], 0.0)


def kernel(idx, A):
    n, n_cols = A.shape
    del idx
    br = next(b for b in (200, 80, 40, 16, 8, 1) if n % b == 0)
    body = functools.partial(_body, n_cols=n_cols, k=K_C)
    return pl.pallas_call(
        body,
        grid=(n // br,),
        in_specs=[pl.BlockSpec((br, n_cols), lambda i: (i, 0))],
        out_specs=pl.BlockSpec((br, n_cols), lambda i: (i, 0)),
        out_shape=jax.ShapeDtypeStruct((n, n_cols), jnp.float32),
        scratch_shapes=[
            pltpu.VMEM((br, n_cols), jnp.int32),
        ],
    )(A)


# speedup vs baseline: 17.4217x; 1.9862x over previous
import functools

import jax
import jax.numpy as jnp
from jax.experimental import pallas as pl
from jax.experimental.pallas import tpu as pltpu

ALPHA_C = 3.0
K_C = 30
_BIG = 0x7FFFFFFF
_LANE = 128


def _bisect_while(count_ge, k, lo0, hi0):
    def cond(carry):
        lo, hi = carry
        return jnp.max(hi - lo) > 1

    def body(carry):
        lo, hi = carry
        mid = lo + ((hi - lo) >> 1)
        ok = count_ge(mid) >= k
        return (jnp.where(ok, mid, lo), jnp.where(ok, hi, mid))

    lo, _ = jax.lax.while_loop(cond, body, (lo0, hi0))
    return lo


def _body(a_ref, o_ref, u_ref, *, n_cols, k):
    x = a_ref[...]
    adj = jnp.tanh(ALPHA_C * x)
    o_ref[...] = adj
    u = jax.lax.bitcast_convert_type(jnp.abs(adj), jnp.int32)
    br = x.shape[0]
    n_pad = u_ref.shape[1]
    u_ref[:, n_cols:n_pad] = jnp.zeros((br, n_pad - n_cols), jnp.int32)
    u_ref[:, 0:n_cols] = u

    mx = u_ref[:, 0:_LANE]
    for i in range(1, n_pad // _LANE):
        mx = jnp.maximum(mx, u_ref[:, i * _LANE:(i + 1) * _LANE])
    rowmax = jnp.max(mx, axis=1, keepdims=True)
    rowmin = jnp.min(mx, axis=1, keepdims=True)

    def count_mx(t):
        return jnp.sum((mx >= t).astype(jnp.int32), axis=1, keepdims=True)

    m_k = _bisect_while(count_mx, k, rowmin, rowmax + 1)

    def count_u(t):
        return jnp.sum((u_ref[...] >= t).astype(jnp.int32), axis=1,
                       keepdims=True)

    t_bits = _bisect_while(count_u, k, m_k, rowmax + 1)

    uu = u_ref[...]
    iota = jax.lax.broadcasted_iota(jnp.int32, (br, n_pad), 1)
    u_ref[...] = jnp.where(uu > t_bits, -1,
                           jnp.where(uu == t_bits, iota, _BIG))

    def count_le(t):
        return jnp.sum((u_ref[...] <= t).astype(jnp.int32), axis=1,
                       keepdims=True)

    def jcond(carry):
        lo, hi = carry
        return jnp.max(hi - lo) > 1

    def jbody(carry):
        lo, hi = carry
        mid = lo + ((hi - lo) >> 1)
        ok = count_le(mid) >= k
        return (jnp.where(ok, lo, mid), jnp.where(ok, mid, hi))

    lo0j = jnp.full((br, 1), -2, jnp.int32)
    hi0j = jnp.full((br, 1), n_cols - 1, jnp.int32)
    _, j_cut = jax.lax.while_loop(jcond, jbody, (lo0j, hi0j))

    keep = u_ref[:, 0:n_cols] <= j_cut
    o_ref[...] = jnp.where(keep, o_ref[...], 0.0)


def kernel(idx, A):
    n, n_cols = A.shape
    del idx
    br = next(b for b in (200, 80, 40, 16, 8, 1) if n % b == 0)
    n_pad = ((n_cols + _LANE - 1) // _LANE) * _LANE
    if n_pad % 256:
        n_pad += _LANE
    body = functools.partial(_body, n_cols=n_cols, k=K_C)
    return pl.pallas_call(
        body,
        grid=(n // br,),
        in_specs=[pl.BlockSpec((br, n_cols), lambda i: (i, 0))],
        out_specs=pl.BlockSpec((br, n_cols), lambda i: (i, 0)),
        out_shape=jax.ShapeDtypeStruct((n, n_cols), jnp.float32),
        scratch_shapes=[
            pltpu.VMEM((br, n_pad), jnp.int32),
        ],
    )(A)


# speedup vs baseline: 18.5412x; 1.0643x over previous
import functools

import jax
import jax.numpy as jnp
from jax.experimental import pallas as pl
from jax.experimental.pallas import tpu as pltpu

ALPHA_C = 3.0
K_C = 30
_BIG = 0x7FFFFFFF
_LANE = 128


def _bisect_while(count_ge, k, lo0, hi0):
    def cond(carry):
        lo, hi = carry
        return jnp.max(hi - lo) > 1

    def step(carry):
        lo, hi = carry
        mid = lo + ((hi - lo) >> 1)
        ok = count_ge(mid) >= k
        return (jnp.where(ok, mid, lo), jnp.where(ok, hi, mid))

    def body(carry):
        return step(step(carry))

    lo, _ = jax.lax.while_loop(cond, body, (lo0, hi0))
    return lo


def _body(a_ref, o_ref, u_ref, *, n_cols, k):
    x = a_ref[...]
    adj = jnp.tanh(ALPHA_C * x)
    o_ref[...] = adj
    u = jax.lax.bitcast_convert_type(jnp.abs(adj), jnp.int32)
    br = x.shape[0]
    n_pad = u_ref.shape[1]
    u_ref[:, n_cols:n_pad] = jnp.zeros((br, n_pad - n_cols), jnp.int32)
    u_ref[:, 0:n_cols] = u

    mx = u_ref[:, 0:_LANE]
    for i in range(1, n_pad // _LANE):
        mx = jnp.maximum(mx, u_ref[:, i * _LANE:(i + 1) * _LANE])
    rowmax = jnp.max(mx, axis=1, keepdims=True)
    rowmin = jnp.min(mx, axis=1, keepdims=True)

    def count_mx(t):
        return jnp.sum((mx >= t).astype(jnp.int32), axis=1, keepdims=True)

    m_k = _bisect_while(count_mx, k, rowmin, rowmax + 1)

    def count_u(t):
        return jnp.sum((u_ref[...] >= t).astype(jnp.int32), axis=1,
                       keepdims=True)

    t_bits = _bisect_while(count_u, k, m_k, rowmax + 1)

    uu = u_ref[...]
    iota = jax.lax.broadcasted_iota(jnp.int32, (br, n_pad), 1)
    u_ref[...] = jnp.where(uu > t_bits, -1,
                           jnp.where(uu == t_bits, iota, _BIG))

    def count_le(t):
        return jnp.sum((u_ref[...] <= t).astype(jnp.int32), axis=1,
                       keepdims=True)

    def jbody(_, carry):
        lo, hi = carry
        mid = lo + ((hi - lo) >> 1)
        ok = count_le(mid) >= k
        return (jnp.where(ok, lo, mid), jnp.where(ok, mid, hi))

    lo0j = jnp.full((br, 1), -2, jnp.int32)
    hi0j = jnp.full((br, 1), n_cols - 1, jnp.int32)
    _, j_cut = jax.lax.fori_loop(0, 14, jbody, (lo0j, hi0j))

    keep = u_ref[:, 0:n_cols] <= j_cut
    o_ref[...] = jnp.where(keep, o_ref[...], 0.0)


def kernel(idx, A):
    n, n_cols = A.shape
    del idx
    br = next(b for b in (200, 80, 40, 16, 8, 1) if n % b == 0)
    n_pad = ((n_cols + _LANE - 1) // _LANE) * _LANE
    if n_pad % 256:
        n_pad += _LANE
    body = functools.partial(_body, n_cols=n_cols, k=K_C)
    return pl.pallas_call(
        body,
        grid=(n // br,),
        in_specs=[pl.BlockSpec((br, n_cols), lambda i: (i, 0))],
        out_specs=pl.BlockSpec((br, n_cols), lambda i: (i, 0)),
        out_shape=jax.ShapeDtypeStruct((n, n_cols), jnp.float32),
        scratch_shapes=[
            pltpu.VMEM((br, n_pad), jnp.int32),
        ],
    )(A)
